# baseline (speedup 1.0000x reference)
import functools

import jax
import jax.numpy as jnp
from jax import lax
from jax.experimental import pallas as pl
from jax.experimental.pallas import tpu as pltpu
from jax.experimental.pallas import tpu_sc as plsc

E = 320000
D = 128
N = 10000
NC = 2
NS = 16
NW = NC * NS
EW = E // NW
B = 100
K = EW // B
RZ = 640
NP = NS * RZ
CW = 16
ZB = 64

_mesh = plsc.VectorSubcoreMesh(
    core_axis_name="c", subcore_axis_name="s", num_cores=NC, num_subcores=NS
)


@functools.partial(
    pl.kernel,
    out_type=(
        jax.ShapeDtypeStruct((NC, NP, D), jnp.float32),
        jax.ShapeDtypeStruct((NC, NP, CW), jnp.float32),
    ),
    mesh=_mesh,
    compiler_params=pltpu.CompilerParams(use_tc_tiling_on_sc=False),
    scratch_types=[
        pltpu.VMEM((K, B), jnp.int32),
        pltpu.VMEM((B, D), jnp.float32),
        pltpu.VMEM((B, D), jnp.float32),
        pltpu.VMEM((B, CW), jnp.float32),
        pltpu.SemaphoreType.DMA,
        pltpu.SemaphoreType.DMA,
        pltpu.SemaphoreType.DMA,
        pltpu.SemaphoreType.DMA,
        pltpu.SemaphoreType.DMA,
        pltpu.VMEM_SHARED((NP, D), jnp.float32),
        pltpu.VMEM_SHARED((NP, CW), jnp.float32),
    ],
)
def _sc_partials(x_hbm, idx_hbm, ones_hbm, zs_hbm, zc_hbm,
                 ps_hbm, pc_hbm, idxv, xbuf0, xbuf1, obuf,
                 lsem0, lsem1, ssem0, ssem1, csem, acc, cnt):
    c = lax.axis_index("c")
    s = lax.axis_index("s")
    wid = c * NS + s
    pltpu.sync_copy(zs_hbm, xbuf0.at[pl.ds(0, ZB)])
    pltpu.sync_copy(zc_hbm, obuf.at[pl.ds(0, ZB)])
    for j in range(RZ // ZB):
        pltpu.sync_copy(xbuf0.at[pl.ds(0, ZB)], acc.at[pl.ds(s * RZ + j * ZB, ZB)])
        pltpu.sync_copy(obuf.at[pl.ds(0, ZB)], cnt.at[pl.ds(s * RZ + j * ZB, ZB)])
    pltpu.sync_copy(idx_hbm.at[wid], idxv)
    pltpu.sync_copy(ones_hbm, obuf)
    plsc.subcore_barrier()

    ebase = wid * EW
    bufs = (xbuf0, xbuf1)
    lsems = (lsem0, lsem1)
    ssems = (ssem0, ssem1)

    def fire_load(k, b):
        pltpu.async_copy(x_hbm.at[pl.ds(ebase + k * B, B)], bufs[b], lsems[b])

    def wait_load(k, b):
        pltpu.make_async_copy(x_hbm.at[pl.ds(ebase + k * B, B)], bufs[b],
                              lsems[b]).wait()

    def fire_sc(k, b):
        pltpu.async_copy(bufs[b], acc.at[idxv.at[k]], ssems[b], add=True)

    def wait_sc(k, b):
        pltpu.make_async_copy(bufs[b], acc.at[idxv.at[k]], ssems[b]).wait()

    def fire_cnt(k):
        pltpu.async_copy(obuf, cnt.at[idxv.at[k]], csem, add=True)

    def wait_cnt(k):
        pltpu.make_async_copy(obuf, cnt.at[idxv.at[k]], csem).wait()

    fire_load(0, 0)
    wait_load(0, 0); fire_sc(0, 0); fire_cnt(0); fire_load(1, 1)
    wait_load(1, 1); fire_sc(1, 1); wait_cnt(0); fire_cnt(1)
    wait_sc(0, 0); fire_load(2, 0)

    def group(g, carry):
        for b in range(2):
            k = 2 * g + b
            wait_load(k, b)
            fire_sc(k, b)
            wait_cnt(k - 1)
            fire_cnt(k)
            wait_sc(k - 1, 1 - b)
            fire_load(k + 1, 1 - b)
        return carry

    lax.fori_loop(1, K // 2 - 1, group, 0)

    k = K - 2
    wait_load(k, 0); fire_sc(k, 0); wait_cnt(k - 1); fire_cnt(k)
    wait_sc(k - 1, 1); fire_load(k + 1, 1)
    k = K - 1
    wait_load(k, 1); fire_sc(k, 1); wait_cnt(k - 1); fire_cnt(k)
    wait_sc(k - 1, 0)
    wait_sc(K - 1, 1)
    wait_cnt(K - 1)

    plsc.subcore_barrier()
    for j in range(RZ // ZB):
        pltpu.sync_copy(acc.at[pl.ds(s * RZ + j * ZB, ZB)], xbuf0.at[pl.ds(0, ZB)])
        pltpu.sync_copy(xbuf0.at[pl.ds(0, ZB)], ps_hbm.at[c, pl.ds(s * RZ + j * ZB, ZB)])
        pltpu.sync_copy(cnt.at[pl.ds(s * RZ + j * ZB, ZB)], obuf.at[pl.ds(0, ZB)])
        pltpu.sync_copy(obuf.at[pl.ds(0, ZB)], pc_hbm.at[c, pl.ds(s * RZ + j * ZB, ZB)])


ROWS_BLK = 640


def _combine_body(ps_ref, pc_ref, o_ref):
    ssum = ps_ref[0] + ps_ref[1]
    csum = pc_ref[0] + pc_ref[1]
    o_ref[...] = ssum / jnp.maximum(csum[:, 0:1], 1.0)


_combine = pl.pallas_call(
    _combine_body,
    grid=(NP // ROWS_BLK,),
    in_specs=[
        pl.BlockSpec((NC, ROWS_BLK, D), lambda i: (0, i, 0)),
        pl.BlockSpec((NC, ROWS_BLK, CW), lambda i: (0, i, 0)),
    ],
    out_specs=pl.BlockSpec((ROWS_BLK, D), lambda i: (i, 0)),
    out_shape=jax.ShapeDtypeStruct((NP, D), jnp.float32),
)


def kernel(x, index, dim_size):
    del dim_size
    idx3d = index.astype(jnp.int32).reshape(NW, K, B)
    ones = jnp.ones((B, CW), jnp.float32)
    zs = jnp.zeros((ZB, D), jnp.float32)
    zc = jnp.zeros((ZB, CW), jnp.float32)
    ps, pc = _sc_partials(x, idx3d, ones, zs, zc)
    return _combine(ps, pc)[:N]


# speedup vs baseline: 1.1709x; 1.1709x over previous
import functools

import jax
import jax.numpy as jnp
from jax import lax
from jax.experimental import pallas as pl
from jax.experimental.pallas import tpu as pltpu
from jax.experimental.pallas import tpu_sc as plsc

E = 320000
D = 128
N = 10000
NC = 2
NS = 16
NW = NC * NS
EW = E // NW
B = 100
K = EW // B
RZ = 640
NP = NS * RZ
CW = 8
ZB = 64
NZ = RZ // ZB

_mesh = plsc.VectorSubcoreMesh(
    core_axis_name="c", subcore_axis_name="s", num_cores=NC, num_subcores=NS
)


@functools.partial(
    pl.kernel,
    out_type=(
        jax.ShapeDtypeStruct((NC, NP, D), jnp.float32),
        jax.ShapeDtypeStruct((NC, NP, CW), jnp.float32),
    ),
    mesh=_mesh,
    compiler_params=pltpu.CompilerParams(use_tc_tiling_on_sc=False),
    scratch_types=[
        pltpu.VMEM((K, B), jnp.int32),
        pltpu.VMEM((B, D), jnp.float32),
        pltpu.VMEM((B, D), jnp.float32),
        pltpu.VMEM((B, CW), jnp.float32),
        pltpu.VMEM((ZB, CW), jnp.float32),
        pltpu.VMEM((ZB, CW), jnp.float32),
        pltpu.SemaphoreType.DMA,
        pltpu.SemaphoreType.DMA,
        pltpu.SemaphoreType.DMA,
        pltpu.SemaphoreType.DMA,
        pltpu.SemaphoreType.DMA,
        pltpu.VMEM_SHARED((NP, D), jnp.float32),
        pltpu.VMEM_SHARED((NP, CW), jnp.float32),
    ],
)
def _sc_partials(x_hbm, idx_hbm, ones_hbm, zs_hbm, zc_hbm,
                 ps_hbm, pc_hbm, idxv, xbuf0, xbuf1, obuf, cbuf0, cbuf1,
                 lsem0, lsem1, csem, asemA, asemB, acc, cnt):
    c = lax.axis_index("c")
    s = lax.axis_index("s")
    wid = c * NS + s
    pltpu.sync_copy(zs_hbm, xbuf0.at[pl.ds(0, ZB)])
    pltpu.sync_copy(zc_hbm, cbuf0)
    for j in range(NZ):
        pltpu.async_copy(xbuf0.at[pl.ds(0, ZB)],
                         acc.at[pl.ds(s * RZ + j * ZB, ZB)], asemA)
        pltpu.async_copy(cbuf0, cnt.at[pl.ds(s * RZ + j * ZB, ZB)], asemB)
    pltpu.sync_copy(idx_hbm.at[wid], idxv)
    pltpu.sync_copy(ones_hbm, obuf)
    for j in range(NZ):
        pltpu.make_async_copy(xbuf0.at[pl.ds(0, ZB)],
                              acc.at[pl.ds(s * RZ + j * ZB, ZB)], asemA).wait()
        pltpu.make_async_copy(cbuf0, cnt.at[pl.ds(s * RZ + j * ZB, ZB)],
                              asemB).wait()
    plsc.subcore_barrier()

    ebase = wid * EW
    bufs = (xbuf0, xbuf1)
    lsems = (lsem0, lsem1)

    def fire_load(k, b):
        pltpu.async_copy(x_hbm.at[pl.ds(ebase + k * B, B)], bufs[b], lsems[b])

    def wait_load(k, b):
        pltpu.make_async_copy(x_hbm.at[pl.ds(ebase + k * B, B)], bufs[b],
                              lsems[b]).wait()

    def fire_cnt(k):
        pltpu.async_copy(obuf, cnt.at[idxv.at[k]], csem, add=True)

    def wait_cnt(k):
        pltpu.make_async_copy(obuf, cnt.at[idxv.at[k]], csem).wait()

    fire_load(0, 0)
    fire_load(1, 1)
    wait_load(0, 0)
    pltpu.sync_copy(xbuf0, acc.at[idxv.at[0]], add=True)
    fire_cnt(0)
    fire_load(2, 0)

    def group(g, carry):
        for b, dk in ((1, 1), (0, 2)):
            k = 2 * g + dk
            wait_load(k, b)
            pltpu.sync_copy(bufs[b], acc.at[idxv.at[k]], add=True)
            wait_cnt(k - 1)
            fire_cnt(k)
            fire_load(k + 2, b)
        return carry

    lax.fori_loop(0, K // 2 - 2, group, 0)

    k = K - 3
    wait_load(k, 1)
    pltpu.sync_copy(xbuf1, acc.at[idxv.at[k]], add=True)
    wait_cnt(k - 1)
    fire_cnt(k)
    fire_load(K - 1, 1)
    for k in (K - 2, K - 1):
        b = k % 2
        wait_load(k, b)
        pltpu.sync_copy(bufs[b], acc.at[idxv.at[k]], add=True)
        wait_cnt(k - 1)
        fire_cnt(k)
    wait_cnt(K - 1)

    plsc.subcore_barrier()

    xbufs = (xbuf0, xbuf1)
    cbufs = (cbuf0, cbuf1)

    def rd(j, b):
        off = s * RZ + j * ZB
        pltpu.async_copy(acc.at[pl.ds(off, ZB)], xbufs[b].at[pl.ds(0, ZB)], asemA)
        pltpu.async_copy(cnt.at[pl.ds(off, ZB)], cbufs[b], asemB)

    def wait_rd(j, b):
        off = s * RZ + j * ZB
        pltpu.make_async_copy(acc.at[pl.ds(off, ZB)], xbufs[b].at[pl.ds(0, ZB)],
                              asemA).wait()
        pltpu.make_async_copy(cnt.at[pl.ds(off, ZB)], cbufs[b], asemB).wait()

    def wr(j, b):
        off = s * RZ + j * ZB
        pltpu.async_copy(xbufs[b].at[pl.ds(0, ZB)], ps_hbm.at[c, pl.ds(off, ZB)],
                         lsems[b])
        pltpu.async_copy(cbufs[b], pc_hbm.at[c, pl.ds(off, ZB)], csem)

    def wait_wr(j, b):
        off = s * RZ + j * ZB
        pltpu.make_async_copy(xbufs[b].at[pl.ds(0, ZB)],
                              ps_hbm.at[c, pl.ds(off, ZB)], lsems[b]).wait()
        pltpu.make_async_copy(cbufs[b], pc_hbm.at[c, pl.ds(off, ZB)],
                              csem).wait()

    rd(0, 0)
    wait_rd(0, 0)
    wr(0, 0)
    rd(1, 1)
    for j in range(1, NZ):
        b = j % 2
        wait_rd(j, b)
        wr(j, b)
        wait_wr(j - 1, 1 - b)
        if j + 1 < NZ:
            rd(j + 1, 1 - b)
    wait_wr(NZ - 1, (NZ - 1) % 2)


ROWS_BLK = 640


def _combine_body(ps_ref, pc_ref, o_ref):
    ssum = ps_ref[0] + ps_ref[1]
    csum = pc_ref[0] + pc_ref[1]
    o_ref[...] = ssum / jnp.maximum(csum[:, 0:1], 1.0)


_combine = pl.pallas_call(
    _combine_body,
    grid=(NP // ROWS_BLK,),
    in_specs=[
        pl.BlockSpec((NC, ROWS_BLK, D), lambda i: (0, i, 0)),
        pl.BlockSpec((NC, ROWS_BLK, CW), lambda i: (0, i, 0)),
    ],
    out_specs=pl.BlockSpec((ROWS_BLK, D), lambda i: (i, 0)),
    out_shape=jax.ShapeDtypeStruct((NP, D), jnp.float32),
)


def kernel(x, index, dim_size):
    del dim_size
    idx3d = index.astype(jnp.int32).reshape(NW, K, B)
    ones = jnp.ones((B, CW), jnp.float32)
    zs = jnp.zeros((ZB, D), jnp.float32)
    zc = jnp.zeros((ZB, CW), jnp.float32)
    ps, pc = _sc_partials(x, idx3d, ones, zs, zc)
    return _combine(ps, pc)[:N]
